# baseline (speedup 1.0000x reference)
import functools

import jax
import jax.numpy as jnp
from jax import lax
from jax.experimental import pallas as pl
from jax.experimental.pallas import tpu as pltpu
from jax.experimental.pallas import tpu_sc as plsc

N_TOK = 4096
D_IN = 1024
N_FEAT = 16384
TOPK = 32


_TM_A = 512
_TF_A = 1024


def _enc_body(x_ref, we_ref, be_ref, h_ref):
    h = jax.lax.dot_general(
        x_ref[...], we_ref[...],
        (((1,), (1,)), ((), ())),
        preferred_element_type=jnp.float32,
        precision=jax.lax.Precision.DEFAULT,
    )
    h_ref[...] = h + be_ref[...][None, :]


def _encoder(x_bf16, W_e_bf16, b_e):
    grid = (N_FEAT // _TF_A, N_TOK // _TM_A)
    return pl.pallas_call(
        _enc_body,
        grid=grid,
        in_specs=[
            pl.BlockSpec((_TM_A, D_IN), lambda f, t: (t, 0)),
            pl.BlockSpec((_TF_A, D_IN), lambda f, t: (f, 0)),
            pl.BlockSpec((_TF_A,), lambda f, t: (f,)),
        ],
        out_specs=pl.BlockSpec((_TM_A, _TF_A), lambda f, t: (t, f)),
        out_shape=jax.ShapeDtypeStruct((N_TOK, N_FEAT), jnp.float32),
    )(x_bf16, W_e_bf16, b_e)


_TM_B = 128
_BISECT_ITERS = 26


def _sel_body(h_ref, mask_ref, out_ref, lo_ref, hi_ref, done_ref):
    h = h_ref[...]

    cols = [
        jnp.max(h[:, c * 128:(c + 1) * 128], axis=1, keepdims=True)
        for c in range(N_FEAT // 128)
    ]
    cm = jnp.concatenate(cols, axis=1)

    gmax = jnp.max(cm, axis=1, keepdims=True)
    lo0 = jnp.min(cm, axis=1, keepdims=True) - 1.0

    def cm_body(_, carry):
        lo, hi = carry
        m = 0.5 * (lo + hi)
        cnt = jnp.sum((cm > m).astype(jnp.float32), axis=1, keepdims=True)
        pred = cnt >= (TOPK + 1)
        return jnp.where(pred, m, lo), jnp.where(pred, hi, m)

    t0, _ = jax.lax.fori_loop(0, 24, cm_body, (lo0, gmax))

    lo_ref[...] = t0
    hi_ref[...] = gmax
    done_ref[...] = jnp.zeros_like(gmax)

    ones8 = jnp.ones((N_FEAT, 8), jnp.bfloat16)
    one_b = jnp.bfloat16(1.0)
    zero_b = jnp.bfloat16(0.0)

    def body(i, _):
        alldone = jnp.min(done_ref[...]) > 0.5

        @pl.when(jnp.logical_not(alldone))
        def _():
            lo = lo_ref[...]
            hi = hi_ref[...]
            done = done_ref[...] > 0.5
            m = 0.5 * (lo + hi)
            ind = jnp.where(h > m, 1.0, 0.0).astype(jnp.bfloat16)
            cnt = jax.lax.dot_general(
                ind, ones8,
                (((1,), (0,)), ((), ())),
                preferred_element_type=jnp.float32,
            )[:, :1]
            pred = cnt >= (TOPK + 1.0)
            live = jnp.logical_not(done)
            lo_ref[...] = jnp.where(live & pred, m, lo)
            hi_ref[...] = jnp.where(live & jnp.logical_not(pred), m, hi)
            done_ref[...] = (done | (cnt == float(TOPK))).astype(jnp.float32)

        return 0

    jax.lax.fori_loop(0, _BISECT_ITERS, body, 0)
    hi = hi_ref[...]
    keep = h > hi
    mask_ref[...] = keep.astype(jnp.float32)
    out_ref[...] = jnp.where(keep & (h > 0.0), h, 0.0)


def _select(h):
    grid = (N_TOK // _TM_B,)
    return pl.pallas_call(
        _sel_body,
        grid=grid,
        in_specs=[pl.BlockSpec((_TM_B, N_FEAT), lambda t: (t, 0))],
        out_specs=[
            pl.BlockSpec((_TM_B, N_FEAT), lambda t: (t, 0)),
            pl.BlockSpec((_TM_B, N_FEAT), lambda t: (t, 0)),
        ],
        out_shape=[
            jax.ShapeDtypeStruct((N_TOK, N_FEAT), jnp.float32),
            jax.ShapeDtypeStruct((N_TOK, N_FEAT), jnp.float32),
        ],
        scratch_shapes=[
            pltpu.VMEM((_TM_B, 1), jnp.float32),
            pltpu.VMEM((_TM_B, 1), jnp.float32),
            pltpu.VMEM((_TM_B, 1), jnp.float32),
        ],
    )(h)


_NW = 32
_RPW = N_TOK // _NW
_NV = N_FEAT // 16
_NBLK = 16
_VPB = _NV // _NBLK
_NEG = -3.0e38


def _sc_find_row_thr(row_v, cm_v, cand_v):
    neg = jnp.full((16,), _NEG, jnp.float32)

    def blk(b, gmax):
        def inner(j, acc):
            v = row_v[pl.ds((b * _VPB + j) * 16, 16)]
            return jnp.maximum(acc, v)
        acc = lax.fori_loop(0, _VPB, inner, neg)
        cm_v[pl.ds(b * 16, 16)] = acc
        return jnp.maximum(gmax, acc)

    gmax_v = lax.fori_loop(0, _NBLK, blk, neg)
    gmax = jnp.full((16,), jnp.max(gmax_v, axis=0), jnp.float32)

    def bis0(_, c):
        lo, hi = c
        m = 0.5 * (lo + hi)

        def cf(j, acc):
            v = cm_v[pl.ds(j * 16, 16)]
            return acc + plsc.all_reduce_population_count(v > m)

        cnt = lax.fori_loop(0, _NBLK, cf, jnp.zeros((16,), jnp.int32))
        p = cnt >= TOPK + 1
        return jnp.where(p, m, lo), jnp.where(p, hi, m)

    lo0 = jnp.full((16,), jnp.min(gmax_v, axis=0) - 1.0, jnp.float32)
    t0, _ = lax.fori_loop(0, 10, bis0, (lo0, gmax))

    def app(j, off):
        v = row_v[pl.ds(j * 16, 16)]
        m = v > t0
        mi = m.astype(jnp.int32)
        dst = off + plsc.cumsum(mi) - 1
        plsc.store_scatter(cand_v, [dst], v, mask=m)
        return off + plsc.all_reduce_population_count(m)

    off = lax.fori_loop(0, _NV, app, jnp.zeros((16,), jnp.int32))
    off_s = jnp.max(off, axis=0)
    cand_v[pl.ds(off_s, 16)] = neg
    nvec = (off_s + 15) // 16

    def bis1(_, c):
        lo, hi = c
        m = 0.5 * (lo + hi)

        def cf(j, acc):
            v = cand_v[pl.ds(j * 16, 16)]
            return acc + plsc.all_reduce_population_count(v > m)

        cnt = lax.fori_loop(0, nvec, cf, jnp.zeros((16,), jnp.int32))
        p = cnt >= TOPK + 1
        return jnp.where(p, m, lo), jnp.where(p, hi, m)

    _, thr = lax.fori_loop(0, 26, bis1, (t0, gmax))
    return thr


def _sc_select_thr(h):
    mesh = plsc.VectorSubcoreMesh(core_axis_name="c", subcore_axis_name="s")

    @functools.partial(
        pl.kernel,
        mesh=mesh,
        out_type=jax.ShapeDtypeStruct((N_TOK,), jnp.float32),
        scratch_types=[
            pltpu.VMEM((N_FEAT,), jnp.float32),
            pltpu.VMEM((256,), jnp.float32),
            pltpu.VMEM((N_FEAT + 16,), jnp.float32),
            pltpu.VMEM((_RPW,), jnp.float32),
            pltpu.SemaphoreType.DMA,
        ],
    )
    def k(h_hbm, thr_hbm, row_a, cm_v, cand_v, thr_v, sem):
        wid = lax.axis_index("s") * 2 + lax.axis_index("c")
        base = wid * _RPW
        lane = lax.iota(jnp.int32, 16)

        def per_row(r, acc):
            pltpu.async_copy(h_hbm.at[base + r], row_a, sem).wait()
            thr = _sc_find_row_thr(row_a, cm_v, cand_v)
            acc = jnp.where(lane == (r & 15), thr, acc)

            @pl.when(((r + 1) & 15) == 0)
            def _():
                thr_v[pl.ds(r - 15, 16)] = acc

            return acc

        lax.fori_loop(0, _RPW, per_row, jnp.zeros((16,), jnp.float32))
        pltpu.sync_copy(thr_v, thr_hbm.at[pl.ds(base, _RPW)])

    return k(h)


def _mask_body(h_ref, thr_ref, mask_ref, out_ref):
    h = h_ref[...]
    thr = thr_ref[...][:, None]
    keep = h > thr
    mask_ref[...] = keep.astype(jnp.float32)
    out_ref[...] = jnp.where(keep & (h > 0.0), h, 0.0)


def _mask_out(h, thr):
    grid = (N_TOK // _TM_B,)
    return pl.pallas_call(
        _mask_body,
        grid=grid,
        in_specs=[
            pl.BlockSpec((_TM_B, N_FEAT), lambda t: (t, 0)),
            pl.BlockSpec((_TM_B,), lambda t: (t,)),
        ],
        out_specs=[
            pl.BlockSpec((_TM_B, N_FEAT), lambda t: (t, 0)),
            pl.BlockSpec((_TM_B, N_FEAT), lambda t: (t, 0)),
        ],
        out_shape=[
            jax.ShapeDtypeStruct((N_TOK, N_FEAT), jnp.float32),
            jax.ShapeDtypeStruct((N_TOK, N_FEAT), jnp.float32),
        ],
    )(h, thr)


_TM_C = 128


def _dec_body(out_ref, wd_ref, bd_ref, recon_ref):
    ob = out_ref[...].astype(jnp.bfloat16)
    wb = wd_ref[...]
    r = jax.lax.dot_general(
        ob, wb,
        (((1,), (1,)), ((), ())),
        preferred_element_type=jnp.float32,
    )
    recon_ref[...] = r + bd_ref[...][None, :]


def _decoder(out, W_d_bf16, b_d):
    grid = (N_TOK // _TM_C,)
    return pl.pallas_call(
        _dec_body,
        grid=grid,
        in_specs=[
            pl.BlockSpec((_TM_C, N_FEAT), lambda t: (t, 0)),
            pl.BlockSpec((D_IN, N_FEAT), lambda t: (0, 0)),
            pl.BlockSpec((D_IN,), lambda t: (0,)),
        ],
        out_specs=pl.BlockSpec((_TM_C, D_IN), lambda t: (t, 0)),
        out_shape=jax.ShapeDtypeStruct((N_TOK, D_IN), jnp.float32),
    )(out, W_d_bf16, b_d)


def kernel(x, W_e, b_e, W_d, b_d):
    h = _encoder(x.astype(jnp.bfloat16), W_e.astype(jnp.bfloat16), b_e)
    mask, out = _select(h)
    recon = _decoder(out, W_d.astype(jnp.bfloat16), b_d)
    return (recon, out, mask)


# speedup vs baseline: 1.2443x; 1.2443x over previous
import functools

import jax
import jax.numpy as jnp
from jax import lax
from jax.experimental import pallas as pl
from jax.experimental.pallas import tpu as pltpu
from jax.experimental.pallas import tpu_sc as plsc

N_TOK = 4096
D_IN = 1024
N_FEAT = 16384
TOPK = 32


_TM_A = 512
_TF_A = 1024


def _enc_body(x_ref, we_ref, be_ref, h_ref):
    h = jax.lax.dot_general(
        x_ref[...], we_ref[...],
        (((1,), (1,)), ((), ())),
        preferred_element_type=jnp.float32,
        precision=jax.lax.Precision.DEFAULT,
    )
    h_ref[...] = h + be_ref[...][None, :]


def _encoder(x_bf16, W_e_bf16, b_e):
    grid = (N_FEAT // _TF_A, N_TOK // _TM_A)
    return pl.pallas_call(
        _enc_body,
        grid=grid,
        in_specs=[
            pl.BlockSpec((_TM_A, D_IN), lambda f, t: (t, 0)),
            pl.BlockSpec((_TF_A, D_IN), lambda f, t: (f, 0)),
            pl.BlockSpec((_TF_A,), lambda f, t: (f,)),
        ],
        out_specs=pl.BlockSpec((_TM_A, _TF_A), lambda f, t: (t, f)),
        out_shape=jax.ShapeDtypeStruct((N_TOK, N_FEAT), jnp.float32),
    )(x_bf16, W_e_bf16, b_e)


_TM_B = 128
_BISECT_ITERS = 26


def _sel_body(h_ref, mask_ref, out_ref, lo_ref, hi_ref, done_ref):
    h = h_ref[...]

    cols = [
        jnp.max(h[:, c * 128:(c + 1) * 128], axis=1, keepdims=True)
        for c in range(N_FEAT // 128)
    ]
    cm = jnp.concatenate(cols, axis=1)

    gmax = jnp.max(cm, axis=1, keepdims=True)
    lo0 = jnp.min(cm, axis=1, keepdims=True) - 1.0

    def cm_body(_, carry):
        lo, hi = carry
        m = 0.5 * (lo + hi)
        cnt = jnp.sum((cm > m).astype(jnp.float32), axis=1, keepdims=True)
        pred = cnt >= (TOPK + 1)
        return jnp.where(pred, m, lo), jnp.where(pred, hi, m)

    t0, _ = jax.lax.fori_loop(0, 24, cm_body, (lo0, gmax))

    lo_ref[...] = t0
    hi_ref[...] = gmax
    done_ref[...] = jnp.zeros_like(gmax)

    def body(i, _):
        alldone = jnp.min(done_ref[...]) > 0.5

        @pl.when(jnp.logical_not(alldone))
        def _():
            lo = lo_ref[...]
            hi = hi_ref[...]
            done = done_ref[...] > 0.5
            m = 0.5 * (lo + hi)
            cnt = jnp.sum((h > m).astype(jnp.float32), axis=1, keepdims=True)
            pred = cnt >= (TOPK + 1.0)
            live = jnp.logical_not(done)
            lo_ref[...] = jnp.where(live & pred, m, lo)
            hi_ref[...] = jnp.where(live & jnp.logical_not(pred), m, hi)
            done_ref[...] = (done | (cnt == float(TOPK))).astype(jnp.float32)

        return 0

    jax.lax.fori_loop(0, _BISECT_ITERS, body, 0)
    hi = hi_ref[...]
    keep = h > hi
    mask_ref[...] = keep.astype(jnp.float32)
    out_ref[...] = jnp.where(keep & (h > 0.0), h, 0.0)


def _select(h):
    grid = (N_TOK // _TM_B,)
    return pl.pallas_call(
        _sel_body,
        grid=grid,
        in_specs=[pl.BlockSpec((_TM_B, N_FEAT), lambda t: (t, 0))],
        out_specs=[
            pl.BlockSpec((_TM_B, N_FEAT), lambda t: (t, 0)),
            pl.BlockSpec((_TM_B, N_FEAT), lambda t: (t, 0)),
        ],
        out_shape=[
            jax.ShapeDtypeStruct((N_TOK, N_FEAT), jnp.float32),
            jax.ShapeDtypeStruct((N_TOK, N_FEAT), jnp.float32),
        ],
        scratch_shapes=[
            pltpu.VMEM((_TM_B, 1), jnp.float32),
            pltpu.VMEM((_TM_B, 1), jnp.float32),
            pltpu.VMEM((_TM_B, 1), jnp.float32),
        ],
    )(h)


_NW = 32
_RPW = N_TOK // _NW
_NV = N_FEAT // 16
_NBLK = 16
_VPB = _NV // _NBLK
_NEG = -3.0e38


def _sc_find_row_thr(row_v, cm_v, cand_v):
    neg = jnp.full((16,), _NEG, jnp.float32)

    def blk(b, gmax):
        def inner(j, acc):
            v = row_v[pl.ds((b * _VPB + j) * 16, 16)]
            return jnp.maximum(acc, v)
        acc = lax.fori_loop(0, _VPB, inner, neg)
        cm_v[pl.ds(b * 16, 16)] = acc
        return jnp.maximum(gmax, acc)

    gmax_v = lax.fori_loop(0, _NBLK, blk, neg)
    gmax = jnp.full((16,), jnp.max(gmax_v, axis=0), jnp.float32)

    def bis0(_, c):
        lo, hi = c
        m = 0.5 * (lo + hi)

        def cf(j, acc):
            v = cm_v[pl.ds(j * 16, 16)]
            return acc + plsc.all_reduce_population_count(v > m)

        cnt = lax.fori_loop(0, _NBLK, cf, jnp.zeros((16,), jnp.int32))
        p = cnt >= TOPK + 1
        return jnp.where(p, m, lo), jnp.where(p, hi, m)

    lo0 = jnp.full((16,), jnp.min(gmax_v, axis=0) - 1.0, jnp.float32)
    t0, _ = lax.fori_loop(0, 10, bis0, (lo0, gmax))

    def app(j, off):
        v = row_v[pl.ds(j * 16, 16)]
        m = v > t0
        mi = m.astype(jnp.int32)
        dst = off + plsc.cumsum(mi) - 1
        plsc.store_scatter(cand_v, [dst], v, mask=m)
        return off + plsc.all_reduce_population_count(m)

    off = lax.fori_loop(0, _NV, app, jnp.zeros((16,), jnp.int32))
    off_s = jnp.max(off, axis=0)
    cand_v[pl.ds(off_s, 16)] = neg
    nvec = (off_s + 15) // 16

    def bis1(_, c):
        lo, hi = c
        m = 0.5 * (lo + hi)

        def cf(j, acc):
            v = cand_v[pl.ds(j * 16, 16)]
            return acc + plsc.all_reduce_population_count(v > m)

        cnt = lax.fori_loop(0, nvec, cf, jnp.zeros((16,), jnp.int32))
        p = cnt >= TOPK + 1
        return jnp.where(p, m, lo), jnp.where(p, hi, m)

    _, thr = lax.fori_loop(0, 26, bis1, (t0, gmax))
    return thr


def _sc_select_thr(h):
    mesh = plsc.VectorSubcoreMesh(core_axis_name="c", subcore_axis_name="s")

    @functools.partial(
        pl.kernel,
        mesh=mesh,
        out_type=jax.ShapeDtypeStruct((N_TOK,), jnp.float32),
        scratch_types=[
            pltpu.VMEM((N_FEAT,), jnp.float32),
            pltpu.VMEM((256,), jnp.float32),
            pltpu.VMEM((N_FEAT + 16,), jnp.float32),
            pltpu.VMEM((_RPW,), jnp.float32),
            pltpu.SemaphoreType.DMA,
        ],
    )
    def k(h_hbm, thr_hbm, row_a, cm_v, cand_v, thr_v, sem):
        wid = lax.axis_index("s") * 2 + lax.axis_index("c")
        base = wid * _RPW
        lane = lax.iota(jnp.int32, 16)

        def per_row(r, acc):
            pltpu.async_copy(h_hbm.at[base + r], row_a, sem).wait()
            thr = _sc_find_row_thr(row_a, cm_v, cand_v)
            acc = jnp.where(lane == (r & 15), thr, acc)

            @pl.when(((r + 1) & 15) == 0)
            def _():
                thr_v[pl.ds(r - 15, 16)] = acc

            return acc

        lax.fori_loop(0, _RPW, per_row, jnp.zeros((16,), jnp.float32))
        pltpu.sync_copy(thr_v, thr_hbm.at[pl.ds(base, _RPW)])

    return k(h)


def _mask_body(h_ref, thr_ref, mask_ref, out_ref):
    h = h_ref[...]
    thr = thr_ref[...][:, None]
    keep = h > thr
    mask_ref[...] = keep.astype(jnp.float32)
    out_ref[...] = jnp.where(keep & (h > 0.0), h, 0.0)


def _mask_out(h, thr):
    grid = (N_TOK // _TM_B,)
    return pl.pallas_call(
        _mask_body,
        grid=grid,
        in_specs=[
            pl.BlockSpec((_TM_B, N_FEAT), lambda t: (t, 0)),
            pl.BlockSpec((_TM_B,), lambda t: (t,)),
        ],
        out_specs=[
            pl.BlockSpec((_TM_B, N_FEAT), lambda t: (t, 0)),
            pl.BlockSpec((_TM_B, N_FEAT), lambda t: (t, 0)),
        ],
        out_shape=[
            jax.ShapeDtypeStruct((N_TOK, N_FEAT), jnp.float32),
            jax.ShapeDtypeStruct((N_TOK, N_FEAT), jnp.float32),
        ],
    )(h, thr)


_TM_C = 256
_KF_C = 4
_TF_C = N_FEAT // _KF_C


def _dec_body(out_ref, wd_ref, bd_ref, recon_ref, acc_ref):
    f = pl.program_id(0)
    t = pl.program_id(1)
    part = jax.lax.dot_general(
        out_ref[...].astype(jnp.bfloat16), wd_ref[...],
        (((1,), (1,)), ((), ())),
        preferred_element_type=jnp.float32,
    )

    @pl.when(f == 0)
    def _():
        acc_ref[t] = part

    @pl.when(f > 0)
    def _():
        acc_ref[t] += part

    @pl.when(f == _KF_C - 1)
    def _():
        recon_ref[...] = acc_ref[t] + bd_ref[...][None, :]


def _decoder(out, W_d_bf16, b_d):
    grid = (_KF_C, N_TOK // _TM_C)
    return pl.pallas_call(
        _dec_body,
        grid=grid,
        in_specs=[
            pl.BlockSpec((_TM_C, _TF_C), lambda f, t: (t, f)),
            pl.BlockSpec((D_IN, _TF_C), lambda f, t: (0, f)),
            pl.BlockSpec((D_IN,), lambda f, t: (0,)),
        ],
        out_specs=pl.BlockSpec((_TM_C, D_IN), lambda f, t: (t, 0)),
        out_shape=jax.ShapeDtypeStruct((N_TOK, D_IN), jnp.float32),
        scratch_shapes=[
            pltpu.VMEM((N_TOK // _TM_C, _TM_C, D_IN), jnp.float32),
        ],
    )(out, W_d_bf16, b_d)


def _dec_trivial(b_d):
    def bb(bd_ref, r_ref):
        r_ref[...] = jnp.zeros((_TM_C, D_IN), jnp.float32) + bd_ref[...][None, :]

    return pl.pallas_call(
        bb,
        grid=(N_TOK // _TM_C,),
        in_specs=[pl.BlockSpec((D_IN,), lambda t: (0,))],
        out_specs=pl.BlockSpec((_TM_C, D_IN), lambda t: (t, 0)),
        out_shape=jax.ShapeDtypeStruct((N_TOK, D_IN), jnp.float32),
    )(b_d)


def kernel(x, W_e, b_e, W_d, b_d):
    h = _encoder(x.astype(jnp.bfloat16), W_e.astype(jnp.bfloat16), b_e)
    mask, out = _select(h)
    recon = _decoder(out, W_d.astype(jnp.bfloat16), b_d)
    return (recon, out, mask)
